# Initial kernel scaffold; baseline (speedup 1.0000x reference)
import jax
import jax.numpy as jnp
from jax.experimental import pallas as pl


def kernel(hidden_states, cos, sin, Wqkv, Wo, Wgu, Wd, router_w, temperature):
    raise NotImplementedError("write your pallas kernel here")


# speedup vs baseline: 1.7631x; 1.7631x over previous
import functools

import jax
import jax.numpy as jnp
from jax.experimental import pallas as pl
from jax.experimental.pallas import tpu as pltpu

EPS = 1e-5
NEG = -1e30


def _rms(x):
    var = jnp.mean(x * x, axis=-1, keepdims=True)
    return x * jax.lax.rsqrt(var + EPS)


def _router_kernel(hid_ref, rwT_ref, temp_ref, logits_ref, idx_ref, w_ref):
    x = hid_ref[...]
    m = jnp.mean(x, axis=1)
    temp = jnp.clip(temp_ref[0, 0], 0.1, 10.0)
    logits = jnp.dot(m, rwT_ref[...], preferred_element_type=jnp.float32)
    logits = logits / temp
    logits_ref[...] = logits
    B, S = logits.shape
    col = jax.lax.broadcasted_iota(jnp.int32, (B, S), 1)
    m1 = jnp.max(logits, axis=1, keepdims=True)
    i1 = jnp.min(jnp.where(logits == m1, col, S), axis=1, keepdims=True)
    l2 = jnp.where(col == i1, NEG, logits)
    m2 = jnp.max(l2, axis=1, keepdims=True)
    i2 = jnp.min(jnp.where(l2 == m2, col, S), axis=1, keepdims=True)
    idx_ref[...] = jnp.concatenate([i1, i2], axis=1).astype(jnp.int32)
    e2 = jnp.exp(m2 - m1)
    w1 = 1.0 / (1.0 + e2)
    w_ref[...] = jnp.concatenate([w1, 1.0 - w1], axis=1)


def _qkv_kernel(idx_ref, w_ref, x_ref, wqkv_ref, cosf_ref, sA_ref, sB_ref,
                q_ref, kT_ref, v_ref, *, D, hd):
    x = x_ref[0]
    qkv = jnp.dot(x.astype(jnp.bfloat16), wqkv_ref[0],
                  preferred_element_type=jnp.float32)
    q = qkv[:, :D]
    k = qkv[:, D:2 * D]
    v = qkv[:, 2 * D:]
    cosf = cosf_ref[...]
    sA = sA_ref[...]
    sB = sB_ref[...]
    h2 = hd // 2

    def rope(u):
        return (u * cosf + jnp.roll(u, -h2, axis=1) * sA
                + jnp.roll(u, h2, axis=1) * sB)

    q_ref[0] = rope(q).astype(jnp.bfloat16)
    kT_ref[0] = rope(k).astype(jnp.bfloat16).T
    v_ref[0] = v.astype(jnp.bfloat16)


def _attn_kernel(idx_ref, w_ref, q_ref, kT_ref, v_ref, o_ref, *, hd):
    qt = q_ref[0]
    kTt = kT_ref[0]
    vt = v_ref[0]
    scale = 1.0 / (hd ** 0.5)
    outs = []
    for i in range(2):
        qh = qt[:, i * hd:(i + 1) * hd]
        kh = kTt[i * hd:(i + 1) * hd, :]
        vh = vt[:, i * hd:(i + 1) * hd]
        s = jnp.dot(qh, kh, preferred_element_type=jnp.float32) * scale
        m = jnp.max(s, axis=1, keepdims=True)
        p = jnp.exp(s - m)
        p = p * (1.0 / jnp.sum(p, axis=1, keepdims=True))
        ah = jnp.dot(p.astype(jnp.bfloat16), vh,
                     preferred_element_type=jnp.float32)
        outs.append(ah.astype(jnp.bfloat16))
    o_ref[0] = jnp.concatenate(outs, axis=1)


def _wo_kernel(idx_ref, w_ref, a_ref, wo_ref, x_ref, x1_ref):
    o = jnp.dot(a_ref[0], wo_ref[0], preferred_element_type=jnp.float32)
    x1_ref[0] = _rms(x_ref[0] + o)


def _mlp_kernel(idx_ref, w_ref, x1_ref, wgu_ref, wd_ref, o_ref, *, I, K):
    b = pl.program_id(0)
    kk = pl.program_id(2)
    n = b * K + kk
    x1 = x1_ref[0]
    gu = jnp.dot(x1.astype(jnp.bfloat16), wgu_ref[0],
                 preferred_element_type=jnp.float32)
    gate = gu[:, :I]
    up = gu[:, I:]
    h = (gate * jax.lax.logistic(gate)) * up
    mlp = jnp.dot(h.astype(jnp.bfloat16), wd_ref[0],
                  preferred_element_type=jnp.float32)
    y = _rms(x1 + mlp) * w_ref[n]

    @pl.when(kk == 0)
    def _():
        o_ref[0] = y

    @pl.when(kk != 0)
    def _():
        o_ref[0] = o_ref[0] + y


def kernel(hidden_states, cos, sin, Wqkv, Wo, Wgu, Wd, router_w, temperature):
    B, T, D = hidden_states.shape
    S, _, F3 = Wqkv.shape
    I = Wd.shape[1]
    hd = cos.shape[1]
    H = D // hd
    K = 2
    N = B * K

    logits, idxs, ws = pl.pallas_call(
        _router_kernel,
        out_shape=[
            jax.ShapeDtypeStruct((B, S), jnp.float32),
            jax.ShapeDtypeStruct((B, K), jnp.int32),
            jax.ShapeDtypeStruct((B, K), jnp.float32),
        ],
    )(hidden_states, router_w.T, temperature.reshape(1, 1))
    flat_idx = idxs.reshape(-1)
    flat_w = ws.reshape(-1)

    wqkv_b = Wqkv.astype(jnp.bfloat16)
    wo_b = Wo.astype(jnp.bfloat16)
    wgu_b = Wgu.astype(jnp.bfloat16)
    wd_b = Wd.astype(jnp.bfloat16)
    cosf = jnp.tile(cos, (1, H))
    sinf = jnp.tile(sin, (1, H))
    j = jax.lax.broadcasted_iota(jnp.int32, (T, D), 1) % hd
    sA = jnp.where(j < hd // 2, -sinf, 0.0)
    sB = jnp.where(j >= hd // 2, sinf, 0.0)

    TQ = min(512, T)
    grid2 = (N, T // TQ)
    q, kT, v = pl.pallas_call(
        functools.partial(_qkv_kernel, D=D, hd=hd),
        grid_spec=pltpu.PrefetchScalarGridSpec(
            num_scalar_prefetch=2,
            grid=grid2,
            in_specs=[
                pl.BlockSpec((1, TQ, D), lambda n, t, i_, w_: (n // K, t, 0)),
                pl.BlockSpec((1, D, F3), lambda n, t, i_, w_: (i_[n], 0, 0)),
                pl.BlockSpec((TQ, D), lambda n, t, i_, w_: (t, 0)),
                pl.BlockSpec((TQ, D), lambda n, t, i_, w_: (t, 0)),
                pl.BlockSpec((TQ, D), lambda n, t, i_, w_: (t, 0)),
            ],
            out_specs=[
                pl.BlockSpec((1, TQ, D), lambda n, t, i_, w_: (n, t, 0)),
                pl.BlockSpec((1, D, TQ), lambda n, t, i_, w_: (n, 0, t)),
                pl.BlockSpec((1, TQ, D), lambda n, t, i_, w_: (n, t, 0)),
            ],
        ),
        out_shape=[
            jax.ShapeDtypeStruct((N, T, D), jnp.bfloat16),
            jax.ShapeDtypeStruct((N, D, T), jnp.bfloat16),
            jax.ShapeDtypeStruct((N, T, D), jnp.bfloat16),
        ],
        compiler_params=pltpu.CompilerParams(
            dimension_semantics=("parallel", "parallel")),
    )(flat_idx, flat_w, hidden_states, wqkv_b, cosf, sA, sB)

    TQA = min(512, T)
    hp = 2 * hd
    grid3 = (N, H // 2, T // TQA)
    attn = pl.pallas_call(
        functools.partial(_attn_kernel, hd=hd),
        grid_spec=pltpu.PrefetchScalarGridSpec(
            num_scalar_prefetch=2,
            grid=grid3,
            in_specs=[
                pl.BlockSpec((1, TQA, hp), lambda n, h, t, i_, w_: (n, t, h)),
                pl.BlockSpec((1, hp, T), lambda n, h, t, i_, w_: (n, h, 0)),
                pl.BlockSpec((1, T, hp), lambda n, h, t, i_, w_: (n, 0, h)),
            ],
            out_specs=pl.BlockSpec((1, TQA, hp),
                                   lambda n, h, t, i_, w_: (n, t, h)),
        ),
        out_shape=jax.ShapeDtypeStruct((N, T, D), jnp.bfloat16),
        compiler_params=pltpu.CompilerParams(
            dimension_semantics=("parallel", "parallel", "parallel")),
    )(flat_idx, flat_w, q, kT, v)

    TW = min(512, T)
    grid4 = (N, T // TW)
    x1 = pl.pallas_call(
        _wo_kernel,
        grid_spec=pltpu.PrefetchScalarGridSpec(
            num_scalar_prefetch=2,
            grid=grid4,
            in_specs=[
                pl.BlockSpec((1, TW, D), lambda n, t, i_, w_: (n, t, 0)),
                pl.BlockSpec((1, D, D), lambda n, t, i_, w_: (i_[n], 0, 0)),
                pl.BlockSpec((1, TW, D), lambda n, t, i_, w_: (n // K, t, 0)),
            ],
            out_specs=pl.BlockSpec((1, TW, D), lambda n, t, i_, w_: (n, t, 0)),
        ),
        out_shape=jax.ShapeDtypeStruct((N, T, D), jnp.float32),
        compiler_params=pltpu.CompilerParams(
            dimension_semantics=("parallel", "parallel")),
    )(flat_idx, flat_w, attn, wo_b, hidden_states)

    TM = min(512, T)
    grid5 = (B, T // TM, K)
    out = pl.pallas_call(
        functools.partial(_mlp_kernel, I=I, K=K),
        grid_spec=pltpu.PrefetchScalarGridSpec(
            num_scalar_prefetch=2,
            grid=grid5,
            in_specs=[
                pl.BlockSpec((1, TM, D),
                             lambda b, t, kk, i_, w_: (b * K + kk, t, 0)),
                pl.BlockSpec((1, D, 2 * I),
                             lambda b, t, kk, i_, w_: (i_[b * K + kk], 0, 0)),
                pl.BlockSpec((1, I, D),
                             lambda b, t, kk, i_, w_: (i_[b * K + kk], 0, 0)),
            ],
            out_specs=pl.BlockSpec((1, TM, D),
                                   lambda b, t, kk, i_, w_: (b, t, 0)),
        ),
        out_shape=jax.ShapeDtypeStruct((B, T, D), jnp.float32),
        compiler_params=pltpu.CompilerParams(
            dimension_semantics=("parallel", "parallel", "arbitrary")),
    )(flat_idx, flat_w, x1, wgu_b, wd_b)

    return out, logits


# speedup vs baseline: 1.8938x; 1.0741x over previous
import functools

import jax
import jax.numpy as jnp
from jax.experimental import pallas as pl
from jax.experimental.pallas import tpu as pltpu

EPS = 1e-5
NEG = -1e30


def _rms(x):
    var = jnp.mean(x * x, axis=-1, keepdims=True)
    return x * jax.lax.rsqrt(var + EPS)


def _router_kernel(hid_ref, rwT_ref, temp_ref, logits_ref, idx_ref, w_ref):
    x = hid_ref[...]
    m = jnp.mean(x, axis=1)
    temp = jnp.clip(temp_ref[0, 0], 0.1, 10.0)
    logits = jnp.dot(m, rwT_ref[...], preferred_element_type=jnp.float32)
    logits = logits / temp
    logits_ref[...] = logits
    B, S = logits.shape
    col = jax.lax.broadcasted_iota(jnp.int32, (B, S), 1)
    m1 = jnp.max(logits, axis=1, keepdims=True)
    i1 = jnp.min(jnp.where(logits == m1, col, S), axis=1, keepdims=True)
    l2 = jnp.where(col == i1, NEG, logits)
    m2 = jnp.max(l2, axis=1, keepdims=True)
    i2 = jnp.min(jnp.where(l2 == m2, col, S), axis=1, keepdims=True)
    idx_ref[...] = jnp.concatenate([i1, i2], axis=1).astype(jnp.int32)
    e2 = jnp.exp(m2 - m1)
    w1 = 1.0 / (1.0 + e2)
    w_ref[...] = jnp.concatenate([w1, 1.0 - w1], axis=1)


def _qkv_kernel(idx_ref, w_ref, x_ref, wqkv_ref, cosf_ref, sA_ref, sB_ref,
                q_ref, kT_ref, v_ref, *, D, hd):
    x = x_ref[0]
    qkv = jnp.dot(x.astype(jnp.bfloat16), wqkv_ref[0],
                  preferred_element_type=jnp.float32)
    q = qkv[:, :D]
    k = qkv[:, D:2 * D]
    v = qkv[:, 2 * D:]
    cosf = cosf_ref[...]
    sA = sA_ref[...]
    sB = sB_ref[...]
    h2 = hd // 2

    def rope(u):
        return (u * cosf + jnp.roll(u, -h2, axis=1) * sA
                + jnp.roll(u, h2, axis=1) * sB)

    q_ref[0] = (rope(q) * (1.0 / (hd ** 0.5))).astype(jnp.bfloat16)
    kT_ref[0] = rope(k).astype(jnp.bfloat16).T
    v_ref[0] = v.astype(jnp.bfloat16)


def _attn_kernel(idx_ref, w_ref, q_ref, kT_ref, v_ref, o_ref, *, hd):
    qt = q_ref[0]
    kTt = kT_ref[0]
    vt = v_ref[0]
    outs = []
    for i in range(2):
        qh = qt[:, i * hd:(i + 1) * hd]
        kh = kTt[i * hd:(i + 1) * hd, :]
        vh = vt[:, i * hd:(i + 1) * hd]
        s = jnp.dot(qh, kh,
                    preferred_element_type=jnp.float32).astype(jnp.bfloat16)
        m = jnp.max(s, axis=1, keepdims=True)
        p = jnp.exp(s - m)
        r = 1.0 / jnp.sum(p, axis=1, keepdims=True).astype(jnp.float32)
        p = p * r.astype(jnp.bfloat16)
        ah = jnp.dot(p, vh, preferred_element_type=jnp.float32)
        outs.append(ah.astype(jnp.bfloat16))
    o_ref[0] = jnp.concatenate(outs, axis=1)


def _wo_kernel(idx_ref, w_ref, a_ref, wo_ref, x_ref, x1_ref):
    o = jnp.dot(a_ref[0], wo_ref[0], preferred_element_type=jnp.float32)
    x1_ref[0] = _rms(x_ref[0] + o)


def _mlp_kernel(idx_ref, w_ref, x1_ref, wgu_ref, wd_ref, o_ref, *, I, K):
    b = pl.program_id(0)
    kk = pl.program_id(2)
    n = b * K + kk
    x1 = x1_ref[0]
    gu = jnp.dot(x1.astype(jnp.bfloat16), wgu_ref[0],
                 preferred_element_type=jnp.float32)
    gate = gu[:, :I]
    up = gu[:, I:]
    h = (gate * jax.lax.logistic(gate)) * up
    mlp = jnp.dot(h.astype(jnp.bfloat16), wd_ref[0],
                  preferred_element_type=jnp.float32)
    y = _rms(x1 + mlp) * w_ref[n]

    @pl.when(kk == 0)
    def _():
        o_ref[0] = y

    @pl.when(kk != 0)
    def _():
        o_ref[0] = o_ref[0] + y


def kernel(hidden_states, cos, sin, Wqkv, Wo, Wgu, Wd, router_w, temperature):
    B, T, D = hidden_states.shape
    S, _, F3 = Wqkv.shape
    I = Wd.shape[1]
    hd = cos.shape[1]
    H = D // hd
    K = 2
    N = B * K

    logits, idxs, ws = pl.pallas_call(
        _router_kernel,
        out_shape=[
            jax.ShapeDtypeStruct((B, S), jnp.float32),
            jax.ShapeDtypeStruct((B, K), jnp.int32),
            jax.ShapeDtypeStruct((B, K), jnp.float32),
        ],
    )(hidden_states, router_w.T, temperature.reshape(1, 1))
    flat_idx = idxs.reshape(-1)
    flat_w = ws.reshape(-1)

    wqkv_b = Wqkv.astype(jnp.bfloat16)
    wo_b = Wo.astype(jnp.bfloat16)
    wgu_b = Wgu.astype(jnp.bfloat16)
    wd_b = Wd.astype(jnp.bfloat16)
    cosf = jnp.tile(cos, (1, H))
    sinf = jnp.tile(sin, (1, H))
    j = jax.lax.broadcasted_iota(jnp.int32, (T, D), 1) % hd
    sA = jnp.where(j < hd // 2, -sinf, 0.0)
    sB = jnp.where(j >= hd // 2, sinf, 0.0)

    TQ = min(512, T)
    grid2 = (N, T // TQ)
    q, kT, v = pl.pallas_call(
        functools.partial(_qkv_kernel, D=D, hd=hd),
        grid_spec=pltpu.PrefetchScalarGridSpec(
            num_scalar_prefetch=2,
            grid=grid2,
            in_specs=[
                pl.BlockSpec((1, TQ, D), lambda n, t, i_, w_: (n // K, t, 0)),
                pl.BlockSpec((1, D, F3), lambda n, t, i_, w_: (i_[n], 0, 0)),
                pl.BlockSpec((TQ, D), lambda n, t, i_, w_: (t, 0)),
                pl.BlockSpec((TQ, D), lambda n, t, i_, w_: (t, 0)),
                pl.BlockSpec((TQ, D), lambda n, t, i_, w_: (t, 0)),
            ],
            out_specs=[
                pl.BlockSpec((1, TQ, D), lambda n, t, i_, w_: (n, t, 0)),
                pl.BlockSpec((1, D, TQ), lambda n, t, i_, w_: (n, 0, t)),
                pl.BlockSpec((1, TQ, D), lambda n, t, i_, w_: (n, t, 0)),
            ],
        ),
        out_shape=[
            jax.ShapeDtypeStruct((N, T, D), jnp.bfloat16),
            jax.ShapeDtypeStruct((N, D, T), jnp.bfloat16),
            jax.ShapeDtypeStruct((N, T, D), jnp.bfloat16),
        ],
        compiler_params=pltpu.CompilerParams(
            dimension_semantics=("parallel", "parallel")),
    )(flat_idx, flat_w, hidden_states, wqkv_b, cosf, sA, sB)

    TQA = min(1024, T)
    hp = 2 * hd
    grid3 = (N, H // 2, T // TQA)
    attn = pl.pallas_call(
        functools.partial(_attn_kernel, hd=hd),
        grid_spec=pltpu.PrefetchScalarGridSpec(
            num_scalar_prefetch=2,
            grid=grid3,
            in_specs=[
                pl.BlockSpec((1, TQA, hp), lambda n, h, t, i_, w_: (n, t, h)),
                pl.BlockSpec((1, hp, T), lambda n, h, t, i_, w_: (n, h, 0)),
                pl.BlockSpec((1, T, hp), lambda n, h, t, i_, w_: (n, 0, h)),
            ],
            out_specs=pl.BlockSpec((1, TQA, hp),
                                   lambda n, h, t, i_, w_: (n, t, h)),
        ),
        out_shape=jax.ShapeDtypeStruct((N, T, D), jnp.bfloat16),
        compiler_params=pltpu.CompilerParams(
            dimension_semantics=("parallel", "parallel", "parallel")),
    )(flat_idx, flat_w, q, kT, v)

    TW = min(512, T)
    grid4 = (N, T // TW)
    x1 = pl.pallas_call(
        _wo_kernel,
        grid_spec=pltpu.PrefetchScalarGridSpec(
            num_scalar_prefetch=2,
            grid=grid4,
            in_specs=[
                pl.BlockSpec((1, TW, D), lambda n, t, i_, w_: (n, t, 0)),
                pl.BlockSpec((1, D, D), lambda n, t, i_, w_: (i_[n], 0, 0)),
                pl.BlockSpec((1, TW, D), lambda n, t, i_, w_: (n // K, t, 0)),
            ],
            out_specs=pl.BlockSpec((1, TW, D), lambda n, t, i_, w_: (n, t, 0)),
        ),
        out_shape=jax.ShapeDtypeStruct((N, T, D), jnp.float32),
        compiler_params=pltpu.CompilerParams(
            dimension_semantics=("parallel", "parallel")),
    )(flat_idx, flat_w, attn, wo_b, hidden_states)

    TM = min(512, T)
    grid5 = (B, T // TM, K)
    out = pl.pallas_call(
        functools.partial(_mlp_kernel, I=I, K=K),
        grid_spec=pltpu.PrefetchScalarGridSpec(
            num_scalar_prefetch=2,
            grid=grid5,
            in_specs=[
                pl.BlockSpec((1, TM, D),
                             lambda b, t, kk, i_, w_: (b * K + kk, t, 0)),
                pl.BlockSpec((1, D, 2 * I),
                             lambda b, t, kk, i_, w_: (i_[b * K + kk], 0, 0)),
                pl.BlockSpec((1, I, D),
                             lambda b, t, kk, i_, w_: (i_[b * K + kk], 0, 0)),
            ],
            out_specs=pl.BlockSpec((1, TM, D),
                                   lambda b, t, kk, i_, w_: (b, t, 0)),
        ),
        out_shape=jax.ShapeDtypeStruct((B, T, D), jnp.float32),
        compiler_params=pltpu.CompilerParams(
            dimension_semantics=("parallel", "parallel", "arbitrary")),
    )(flat_idx, flat_w, x1, wgu_b, wd_b)

    return out, logits
